# baseline (speedup 1.0000x reference)
import functools

import jax
import jax.numpy as jnp
from jax import lax
from jax.experimental import pallas as pl
from jax.experimental.pallas import tpu as pltpu
from jax.experimental.pallas import tpu_sc as plsc

NNODE = 10000
DMODEL = 128
NCORE = 2
NSUB = 16
NTILE = NCORE * NSUB
CHUNK = 128
BLK = 16
NPAD = 10112
RPT = NPAD // NSUB
_mesh = plsc.VectorSubcoreMesh(core_axis_name="c", subcore_axis_name="s")


DEGW = 128


def _deg_kernel(nch):
    ept = nch * CHUNK

    @functools.partial(
        pl.kernel,
        out_type=jax.ShapeDtypeStruct((NCORE, NPAD, DEGW), jnp.float32),
        mesh=_mesh,
        scratch_types=[
            pltpu.VMEM((CHUNK,), jnp.int32),
            pltpu.VMEM((CHUNK, DEGW), jnp.float32),
            pltpu.VMEM_SHARED((NPAD, DEGW), jnp.float32),
        ],
    )
    def k(dst_hbm, ones_hbm, zeros_hbm, out_hbm, dstv, onesv, acc):
        c = lax.axis_index("c")
        s = lax.axis_index("s")
        wid = c * NSUB + s
        r0 = s * RPT
        pltpu.sync_copy(zeros_hbm.at[pl.ds(r0, RPT)], acc.at[pl.ds(r0, RPT)])
        pltpu.sync_copy(ones_hbm, onesv)
        plsc.subcore_barrier()
        e0 = wid * ept

        def body(i, carry):
            base = pl.multiple_of(e0 + i * CHUNK, CHUNK)
            pltpu.sync_copy(dst_hbm.at[pl.ds(base, CHUNK)], dstv)
            pltpu.sync_copy(onesv, acc.at[dstv], add=True)
            return carry

        lax.fori_loop(0, nch, body, 0)
        plsc.subcore_barrier()
        pltpu.sync_copy(acc.at[pl.ds(r0, RPT)], out_hbm.at[c, pl.ds(r0, RPT)])

    return k


def _tc_red(degp_ref, dis_ref):
    deg = degp_ref[0, 0:NNODE, 0:1] + degp_ref[1, 0:NNODE, 0:1] + 1.0
    dis_ref[...] = lax.rsqrt(deg)


def _agg_kernel2(nch_a, nch_b):
    assert nch_a % BLK == 0 and nch_b % BLK == 0

    @functools.partial(
        pl.kernel,
        out_type=jax.ShapeDtypeStruct((NCORE, NPAD, DMODEL), jnp.float32),
        mesh=_mesh,
        scratch_types=[
            pltpu.VMEM((BLK, CHUNK), jnp.int32),
            pltpu.VMEM((BLK, CHUNK), jnp.int32),
            pltpu.VMEM((CHUNK, DMODEL), jnp.float32),
            pltpu.VMEM((CHUNK, DMODEL), jnp.float32),
            pltpu.VMEM_SHARED((NPAD, DMODEL), jnp.float32),
            pltpu.SemaphoreType.DMA,
            pltpu.SemaphoreType.DMA,
        ],
    )
    def k(g_hbm, src_hbm, dst_hbm, zeros_hbm, out_hbm,
          srcb, dstb, rows0, rows1, acc, sem0, sem1):
        c = lax.axis_index("c")
        s = lax.axis_index("s")
        wid = c * NSUB + s
        r0 = s * RPT
        pltpu.sync_copy(zeros_hbm.at[pl.ds(r0, RPT)], acc.at[pl.ds(r0, RPT)])
        plsc.subcore_barrier()
        ch0 = jnp.where(c == 0, s * nch_a, NSUB * nch_a + s * nch_b)
        n_my = jnp.where(c == 0, nch_a, nch_b)
        rows = (rows0, rows1)
        sems = (sem0, sem1)

        def blk_body(bi, carry):
            cb = pl.multiple_of(ch0 + bi * BLK, BLK)
            pltpu.sync_copy(src_hbm.at[pl.ds(cb, BLK)], srcb)
            pltpu.sync_copy(dst_hbm.at[pl.ds(cb, BLK)], dstb)
            handles = [None] * BLK
            handles[0] = pltpu.async_copy(g_hbm.at[srcb.at[0]], rows[0], sems[0])
            for j in range(BLK):
                b = j % 2
                if j + 1 < BLK:
                    handles[j + 1] = pltpu.async_copy(
                        g_hbm.at[srcb.at[j + 1]], rows[1 - b], sems[1 - b])
                handles[j].wait()
                pltpu.sync_copy(rows[b], acc.at[dstb.at[j]], add=True)
            return carry

        lax.fori_loop(0, n_my // BLK, blk_body, 0)
        plsc.subcore_barrier()
        pltpu.sync_copy(acc.at[pl.ds(r0, RPT)], out_hbm.at[c, pl.ds(r0, RPT)])

    return k


def _tc_a(dis_ref, x_ref, w1_ref, g1_ref):
    h = jnp.dot(x_ref[...], w1_ref[...], preferred_element_type=jnp.float32)
    g1_ref[...] = dis_ref[...] * h


def _tc_b(p_ref, g_ref, dis_ref, b1_ref, w2_ref, g2_ref):
    s = p_ref[0, 0:NNODE, :] + p_ref[1, 0:NNODE, :]
    dis = dis_ref[...]
    z = jnp.maximum(dis * (s + g_ref[...]) + b1_ref[...], 0.0)
    h2 = jnp.dot(z, w2_ref[...], preferred_element_type=jnp.float32)
    g2_ref[...] = dis * h2


def _tc_c(p_ref, g_ref, dis_ref, b2_ref, out_ref):
    s = p_ref[0, 0:NNODE, :] + p_ref[1, 0:NNODE, :]
    out_ref[...] = dis_ref[...] * (s + g_ref[...]) + b2_ref[...]


def kernel(x, edge_index, W1, b1, W2, b2):
    e = edge_index.shape[1]
    grain = NTILE * CHUNK
    nch = (e + grain - 1) // grain
    nch = ((nch + BLK - 1) // BLK) * BLK
    epad = nch * grain
    src = edge_index[0].astype(jnp.int32)
    dst = edge_index[1].astype(jnp.int32)
    pad = epad - e
    src = jnp.concatenate([src, jnp.zeros((pad,), jnp.int32)])
    dst = jnp.concatenate([dst, jnp.full((pad,), NNODE, jnp.int32)])
    src2d = src.reshape(NTILE * nch, CHUNK)
    dst2d = dst.reshape(NTILE * nch, CHUNK)

    zeros_acc = jnp.zeros((NPAD, DMODEL), jnp.float32)

    degp = _deg_kernel(nch)(
        dst, jnp.ones((CHUNK, DEGW), jnp.float32),
        jnp.zeros((NPAD, DEGW), jnp.float32))

    dis = pl.pallas_call(
        _tc_red,
        out_shape=jax.ShapeDtypeStruct((NNODE, 1), jnp.float32),
    )(degp)

    g1 = pl.pallas_call(
        _tc_a,
        out_shape=jax.ShapeDtypeStruct((NNODE, DMODEL), jnp.float32),
    )(dis, x, W1)

    nch_a = 2 * nch
    nch_b = 0
    agg = _agg_kernel2(nch_a, nch_b)
    p1 = agg(g1, src2d, dst2d, zeros_acc)

    g2 = pl.pallas_call(
        _tc_b,
        out_shape=jax.ShapeDtypeStruct((NNODE, DMODEL), jnp.float32),
    )(p1, g1, dis, b1.reshape(1, DMODEL), W2)

    p2 = agg(g2, src2d, dst2d, zeros_acc)

    out = pl.pallas_call(
        _tc_c,
        out_shape=jax.ShapeDtypeStruct((NNODE, DMODEL), jnp.float32),
    )(p2, g2, dis, b2.reshape(1, DMODEL))

    return out


# speedup vs baseline: 3.1666x; 3.1666x over previous
import functools

import jax
import jax.numpy as jnp
from jax import lax
from jax.experimental import pallas as pl
from jax.experimental.pallas import tpu as pltpu
from jax.experimental.pallas import tpu_sc as plsc

NNODE = 10000
DMODEL = 128
NCORE = 2
NSUB = 16
NTILE = NCORE * NSUB
CHUNK = 128
BLK = 16
NPAD = 10112
RPT = NPAD // NSUB
_mesh = plsc.VectorSubcoreMesh(core_axis_name="c", subcore_axis_name="s")


DEGW = 128


def _deg_kernel(nch):
    ept = nch * CHUNK

    @functools.partial(
        pl.kernel,
        out_type=jax.ShapeDtypeStruct((NCORE, NPAD, DEGW), jnp.float32),
        mesh=_mesh,
        scratch_types=[
            pltpu.VMEM((CHUNK,), jnp.int32),
            pltpu.VMEM((CHUNK, DEGW), jnp.float32),
            pltpu.VMEM_SHARED((NPAD, DEGW), jnp.float32),
        ],
    )
    def k(dst_hbm, ones_hbm, zeros_hbm, out_hbm, dstv, onesv, acc):
        c = lax.axis_index("c")
        s = lax.axis_index("s")
        wid = c * NSUB + s
        r0 = s * RPT
        pltpu.sync_copy(zeros_hbm.at[pl.ds(r0, RPT)], acc.at[pl.ds(r0, RPT)])
        pltpu.sync_copy(ones_hbm, onesv)
        plsc.subcore_barrier()
        e0 = wid * ept

        def body(i, carry):
            base = pl.multiple_of(e0 + i * CHUNK, CHUNK)
            pltpu.sync_copy(dst_hbm.at[pl.ds(base, CHUNK)], dstv)
            pltpu.sync_copy(onesv, acc.at[dstv], add=True)
            return carry

        lax.fori_loop(0, nch, body, 0)
        plsc.subcore_barrier()
        pltpu.sync_copy(acc.at[pl.ds(r0, RPT)], out_hbm.at[c, pl.ds(r0, RPT)])

    return k


def _tc_red(degp_ref, dis_ref):
    deg = degp_ref[0, 0:NNODE, 0:1] + degp_ref[1, 0:NNODE, 0:1] + 1.0
    dis_ref[...] = lax.rsqrt(deg)


def _agg_kernel2(nch_a, nch_b):
    assert nch_a % BLK == 0 and nch_b % BLK == 0

    @functools.partial(
        pl.kernel,
        out_type=jax.ShapeDtypeStruct((NCORE, NPAD, DMODEL), jnp.float32),
        mesh=_mesh,
        scratch_types=[
            pltpu.VMEM((BLK, CHUNK), jnp.int32),
            pltpu.VMEM((BLK, CHUNK), jnp.int32),
            pltpu.VMEM((CHUNK, DMODEL), jnp.float32),
            pltpu.VMEM((CHUNK, DMODEL), jnp.float32),
            pltpu.VMEM_SHARED((NPAD, DMODEL), jnp.float32),
            pltpu.SemaphoreType.DMA,
            pltpu.SemaphoreType.DMA,
        ],
    )
    def k(g_hbm, src_hbm, dst_hbm, zeros_hbm, out_hbm,
          srcb, dstb, rows0, rows1, acc, sem0, sem1):
        c = lax.axis_index("c")
        s = lax.axis_index("s")
        wid = c * NSUB + s
        r0 = s * RPT
        pltpu.sync_copy(zeros_hbm.at[pl.ds(r0, RPT)], acc.at[pl.ds(r0, RPT)])
        plsc.subcore_barrier()
        ch0 = jnp.where(c == 0, s * nch_a, NSUB * nch_a + s * nch_b)
        n_my = jnp.where(c == 0, nch_a, nch_b)
        rows = (rows0, rows1)
        sems = (sem0, sem1)

        def blk_body(bi, carry):
            cb = pl.multiple_of(ch0 + bi * BLK, BLK)
            pltpu.sync_copy(src_hbm.at[pl.ds(cb, BLK)], srcb)
            pltpu.sync_copy(dst_hbm.at[pl.ds(cb, BLK)], dstb)
            handles = [None] * BLK
            handles[0] = pltpu.async_copy(g_hbm.at[srcb.at[0]], rows[0], sems[0])
            for j in range(BLK):
                b = j % 2
                if j + 1 < BLK:
                    handles[j + 1] = pltpu.async_copy(
                        g_hbm.at[srcb.at[j + 1]], rows[1 - b], sems[1 - b])
                handles[j].wait()
                pltpu.sync_copy(rows[b], acc.at[dstb.at[j]], add=True)
            return carry

        lax.fori_loop(0, n_my // BLK, blk_body, 0)
        plsc.subcore_barrier()
        pltpu.sync_copy(acc.at[pl.ds(r0, RPT)], out_hbm.at[c, pl.ds(r0, RPT)])

    return k


def _tc_a(dis_ref, x_ref, w1_ref, g1_ref):
    h = jnp.dot(x_ref[...], w1_ref[...], preferred_element_type=jnp.float32)
    g1_ref[...] = dis_ref[...] * h


def _tc_b(p_ref, g_ref, dis_ref, b1_ref, w2_ref, g2_ref):
    s = p_ref[0, 0:NNODE, :] + p_ref[1, 0:NNODE, :]
    dis = dis_ref[...]
    z = jnp.maximum(dis * (s + g_ref[...]) + b1_ref[...], 0.0)
    h2 = jnp.dot(z, w2_ref[...], preferred_element_type=jnp.float32)
    g2_ref[...] = dis * h2


def _tc_c(p_ref, g_ref, dis_ref, b2_ref, out_ref):
    s = p_ref[0, 0:NNODE, :] + p_ref[1, 0:NNODE, :]
    out_ref[...] = dis_ref[...] * (s + g_ref[...]) + b2_ref[...]


def kernel(x, edge_index, W1, b1, W2, b2):
    e = edge_index.shape[1]
    grain = NTILE * CHUNK
    nch = (e + grain - 1) // grain
    nch = ((nch + BLK - 1) // BLK) * BLK
    epad = nch * grain
    src = edge_index[0].astype(jnp.int32)
    dst = edge_index[1].astype(jnp.int32)
    pad = epad - e
    pad_idx = jnp.arange(pad, dtype=jnp.int32)
    src = jnp.concatenate([src, pad_idx % NNODE])
    dst = jnp.concatenate([dst, NNODE + pad_idx % (NPAD - NNODE)])
    src2d = src.reshape(NTILE * nch, CHUNK)
    dst2d = dst.reshape(NTILE * nch, CHUNK)

    zeros_acc = jnp.zeros((NPAD, DMODEL), jnp.float32)

    degp = _deg_kernel(nch)(
        dst, jnp.ones((CHUNK, DEGW), jnp.float32),
        jnp.zeros((NPAD, DEGW), jnp.float32))

    dis = pl.pallas_call(
        _tc_red,
        out_shape=jax.ShapeDtypeStruct((NNODE, 1), jnp.float32),
    )(degp)

    g1 = pl.pallas_call(
        _tc_a,
        out_shape=jax.ShapeDtypeStruct((NNODE, DMODEL), jnp.float32),
    )(dis, x, W1)

    nch_a = nch
    nch_b = nch
    agg = _agg_kernel2(nch_a, nch_b)
    p1 = agg(g1, src2d, dst2d, zeros_acc)

    g2 = pl.pallas_call(
        _tc_b,
        out_shape=jax.ShapeDtypeStruct((NNODE, DMODEL), jnp.float32),
    )(p1, g1, dis, b1.reshape(1, DMODEL), W2)

    p2 = agg(g2, src2d, dst2d, zeros_acc)

    out = pl.pallas_call(
        _tc_c,
        out_shape=jax.ShapeDtypeStruct((NNODE, DMODEL), jnp.float32),
    )(p2, g2, dis, b2.reshape(1, DMODEL))

    return out


# speedup vs baseline: 3.4488x; 1.0891x over previous
import functools

import jax
import jax.numpy as jnp
from jax import lax
from jax.experimental import pallas as pl
from jax.experimental.pallas import tpu as pltpu
from jax.experimental.pallas import tpu_sc as plsc

NNODE = 10000
DMODEL = 128
NCORE = 2
NSUB = 16
NTILE = NCORE * NSUB
CHUNK = 128
BLK = 16
NPAD = 10112
RPT = NPAD // NSUB
_mesh = plsc.VectorSubcoreMesh(core_axis_name="c", subcore_axis_name="s")


DEGW = 128


def _deg_kernel(nch):
    assert nch % BLK == 0

    @functools.partial(
        pl.kernel,
        out_type=jax.ShapeDtypeStruct((NCORE, NPAD, DEGW), jnp.float32),
        mesh=_mesh,
        scratch_types=[
            pltpu.VMEM((BLK, CHUNK), jnp.int32),
            pltpu.VMEM((CHUNK, DEGW), jnp.float32),
            pltpu.VMEM_SHARED((NPAD, DEGW), jnp.float32),
            pltpu.SemaphoreType.DMA,
        ],
    )
    def k(dst_hbm, ones_hbm, zeros_hbm, out_hbm, dstb, onesv, acc, sem):
        c = lax.axis_index("c")
        s = lax.axis_index("s")
        wid = c * NSUB + s
        r0 = s * RPT
        pltpu.sync_copy(zeros_hbm.at[pl.ds(r0, RPT)], acc.at[pl.ds(r0, RPT)])
        pltpu.sync_copy(ones_hbm, onesv)
        plsc.subcore_barrier()
        ch0 = wid * nch

        def blk_body(bi, carry):
            cb = pl.multiple_of(ch0 + bi * BLK, BLK)
            pltpu.sync_copy(dst_hbm.at[pl.ds(cb, BLK)], dstb)
            handles = [
                pltpu.async_copy(onesv, acc.at[dstb.at[j]], sem, add=True)
                for j in range(BLK)
            ]
            for h in handles:
                h.wait()
            return carry

        lax.fori_loop(0, nch // BLK, blk_body, 0)
        plsc.subcore_barrier()
        pltpu.sync_copy(acc.at[pl.ds(r0, RPT)], out_hbm.at[c, pl.ds(r0, RPT)])

    return k


def _tc_red(degp_ref, h_ref, dis_ref, g1_ref):
    deg = degp_ref[0, 0:NNODE, 0:1] + degp_ref[1, 0:NNODE, 0:1] + 1.0
    dis = lax.rsqrt(deg)
    dis_ref[...] = dis
    g1_ref[...] = dis * h_ref[...]


def _agg_kernel2(nch_a, nch_b):
    assert nch_a % BLK == 0 and nch_b % BLK == 0

    @functools.partial(
        pl.kernel,
        out_type=jax.ShapeDtypeStruct((NCORE, NPAD, DMODEL), jnp.float32),
        mesh=_mesh,
        scratch_types=[
            pltpu.VMEM((BLK, CHUNK), jnp.int32),
            pltpu.VMEM((BLK, CHUNK), jnp.int32),
            pltpu.VMEM((CHUNK, DMODEL), jnp.float32),
            pltpu.VMEM((CHUNK, DMODEL), jnp.float32),
            pltpu.VMEM_SHARED((NPAD, DMODEL), jnp.float32),
            pltpu.SemaphoreType.DMA,
            pltpu.SemaphoreType.DMA,
        ],
    )
    def k(g_hbm, src_hbm, dst_hbm, zeros_hbm, out_hbm,
          srcb, dstb, rows0, rows1, acc, sem0, sem1):
        c = lax.axis_index("c")
        s = lax.axis_index("s")
        wid = c * NSUB + s
        r0 = s * RPT
        pltpu.sync_copy(zeros_hbm.at[pl.ds(r0, RPT)], acc.at[pl.ds(r0, RPT)])
        plsc.subcore_barrier()
        ch0 = jnp.where(c == 0, s * nch_a, NSUB * nch_a + s * nch_b)
        n_my = jnp.where(c == 0, nch_a, nch_b)
        rows = (rows0, rows1)
        sems = (sem0, sem1)

        def blk_body(bi, carry):
            cb = pl.multiple_of(ch0 + bi * BLK, BLK)
            pltpu.sync_copy(src_hbm.at[pl.ds(cb, BLK)], srcb)
            pltpu.sync_copy(dst_hbm.at[pl.ds(cb, BLK)], dstb)
            handles = [None] * BLK
            handles[0] = pltpu.async_copy(g_hbm.at[srcb.at[0]], rows[0], sems[0])
            for j in range(BLK):
                b = j % 2
                if j + 1 < BLK:
                    handles[j + 1] = pltpu.async_copy(
                        g_hbm.at[srcb.at[j + 1]], rows[1 - b], sems[1 - b])
                handles[j].wait()
                pltpu.sync_copy(rows[b], acc.at[dstb.at[j]], add=True)
            return carry

        lax.fori_loop(0, n_my // BLK, blk_body, 0)
        plsc.subcore_barrier()
        pltpu.sync_copy(acc.at[pl.ds(r0, RPT)], out_hbm.at[c, pl.ds(r0, RPT)])

    return k


def _tc_mm(x_ref, w1_ref, h_ref):
    h_ref[...] = jnp.dot(x_ref[...], w1_ref[...],
                         preferred_element_type=jnp.float32)


def _tc_b(p_ref, g_ref, dis_ref, b1_ref, w2_ref, g2_ref):
    s = p_ref[0, 0:NNODE, :] + p_ref[1, 0:NNODE, :]
    dis = dis_ref[...]
    z = jnp.maximum(dis * (s + g_ref[...]) + b1_ref[...], 0.0)
    h2 = jnp.dot(z, w2_ref[...], preferred_element_type=jnp.float32)
    g2_ref[...] = dis * h2


def _tc_c(p_ref, g_ref, dis_ref, b2_ref, out_ref):
    s = p_ref[0, 0:NNODE, :] + p_ref[1, 0:NNODE, :]
    out_ref[...] = dis_ref[...] * (s + g_ref[...]) + b2_ref[...]


def kernel(x, edge_index, W1, b1, W2, b2):
    e = edge_index.shape[1]
    grain = NTILE * CHUNK
    nch = (e + grain - 1) // grain
    nch = ((nch + BLK - 1) // BLK) * BLK
    epad = nch * grain
    src = edge_index[0].astype(jnp.int32)
    dst = edge_index[1].astype(jnp.int32)
    pad = epad - e
    pad_idx = jnp.arange(pad, dtype=jnp.int32)
    src = jnp.concatenate([src, pad_idx % NNODE])
    dst = jnp.concatenate([dst, NNODE + pad_idx % (NPAD - NNODE)])
    src2d = src.reshape(NTILE * nch, CHUNK)
    dst2d = dst.reshape(NTILE * nch, CHUNK)

    zeros_acc = jnp.zeros((NPAD, DMODEL), jnp.float32)

    h1 = pl.pallas_call(
        _tc_mm,
        out_shape=jax.ShapeDtypeStruct((NNODE, DMODEL), jnp.float32),
    )(x, W1)

    degp = _deg_kernel(nch)(
        dst2d, jnp.ones((CHUNK, DEGW), jnp.float32),
        jnp.zeros((NPAD, DEGW), jnp.float32))

    dis, g1 = pl.pallas_call(
        _tc_red,
        out_shape=(
            jax.ShapeDtypeStruct((NNODE, 1), jnp.float32),
            jax.ShapeDtypeStruct((NNODE, DMODEL), jnp.float32),
        ),
    )(degp, h1)

    nch_a = nch
    nch_b = nch
    agg = _agg_kernel2(nch_a, nch_b)
    p1 = agg(g1, src2d, dst2d, zeros_acc)

    g2 = pl.pallas_call(
        _tc_b,
        out_shape=jax.ShapeDtypeStruct((NNODE, DMODEL), jnp.float32),
    )(p1, g1, dis, b1.reshape(1, DMODEL), W2)

    p2 = agg(g2, src2d, dst2d, zeros_acc)

    out = pl.pallas_call(
        _tc_c,
        out_shape=jax.ShapeDtypeStruct((NNODE, DMODEL), jnp.float32),
    )(p2, g2, dis, b2.reshape(1, DMODEL))

    return out
